# Initial kernel scaffold; baseline (speedup 1.0000x reference)
import jax
import jax.numpy as jnp
from jax.experimental import pallas as pl


def kernel(x, edge_index_rsr, edge_index_rtr, edge_index_rur, W1r, W1n, b1, g1, be1, W2r, W2n, b2, g2, be2, W3r, W3n, b3, g3, be3, Wl, bl):
    raise NotImplementedError("write your pallas kernel here")


# speedup vs baseline: 3.5310x; 3.5310x over previous
import functools

import jax
import jax.numpy as jnp
from jax import lax
from jax.experimental import pallas as pl
from jax.experimental.pallas import tpu as pltpu
from jax.experimental.pallas import tpu_sc as plsc

_N = 10000
_E = 320000
_LANES = 128
_NTILE = 16
_NCHUNK = 160
_EPT = _NCHUNK * _LANES
_EPAD = _EPT * _NTILE
_RPT = 632
_NPAD = _NTILE * _RPT

_D1, _H1, _H2, _H3, _C = 128, 160, 256, 64, 64
_W1, _W2, _W3 = 80, 96, 48


@functools.lru_cache(maxsize=None)
def _seg_sum(w):
  mesh = plsc.VectorSubcoreMesh(core_axis_name="c", subcore_axis_name="s")

  def body(tab2, src2d, dst2d, zeros, out, src_v, dst_v, rows, acc, sem):
    cid = lax.axis_index("c")
    sid = lax.axis_index("s")
    pltpu.sync_copy(zeros.at[pl.ds(sid * _RPT, _RPT)],
                    acc.at[pl.ds(sid * _RPT, _RPT)])
    pltpu.sync_copy(src2d.at[pl.ds(sid * _NCHUNK, _NCHUNK)], src_v)
    pltpu.sync_copy(dst2d.at[pl.ds(sid * _NCHUNK, _NCHUNK)], dst_v)
    plsc.subcore_barrier()
    tab = tab2.at[cid]

    @pl.loop(0, _NCHUNK)
    def _(j):
      pltpu.async_copy(tab.at[src_v.at[j]], rows, sem).wait()
      pltpu.sync_copy(rows, acc.at[dst_v.at[j]], add=True)

    plsc.subcore_barrier()
    pltpu.sync_copy(acc.at[pl.ds(sid * _RPT, _RPT)],
                    out.at[cid, pl.ds(sid * _RPT, _RPT)])

  return pl.kernel(
      body,
      out_type=jax.ShapeDtypeStruct((2, _NPAD, w), jnp.float32),
      mesh=mesh,
      compiler_params=pltpu.CompilerParams(use_tc_tiling_on_sc=False),
      scratch_types=[
          pltpu.VMEM((_NCHUNK, _LANES), jnp.int32),
          pltpu.VMEM((_NCHUNK, _LANES), jnp.int32),
          pltpu.VMEM((_LANES, w), jnp.float32),
          pltpu.VMEM_SHARED((_NPAD, w), jnp.float32),
          pltpu.SemaphoreType.DMA,
      ],
  )


def _prep_edges(ei):
  pad = _EPAD - _E
  src = jnp.concatenate([ei[0], jnp.zeros((pad,), jnp.int32)])
  dst = jnp.concatenate([ei[1], jnp.full((pad,), _N, jnp.int32)])
  return src.reshape(-1, _LANES), dst.reshape(-1, _LANES)


def _mean_from_acc(acc_ref, w, d):
  aL = acc_ref[0, pl.ds(0, _N), :]
  aR = acc_ref[1, pl.ds(0, _N), :]
  agg = jnp.concatenate([aL, aR[:, :d - w]], axis=1)
  cnt = jnp.maximum(aR[:, d - w:d - w + 1], 1.0)
  return agg / cnt


def _bn_relu(h, g, be):
  h = jnp.maximum(h, 0.0)
  m = jnp.mean(h, axis=0, keepdims=True)
  v = jnp.mean((h - m) ** 2, axis=0, keepdims=True)
  return (h - m) * lax.rsqrt(v + 1e-5) * g + be


def _split_tables(h, w, d):
  rpad = 2 * w - d - 1
  right = jnp.concatenate(
      [h[:, w:d], jnp.full((_N, 1), 1.0, jnp.float32),
       jnp.zeros((_N, rpad), jnp.float32)], axis=1)
  return jnp.stack([h[:, :w], right])


def _dense1_body(x_ref, acc_ref, wr_ref, wn_ref, b_ref, g_ref, be_ref,
                 t2_ref):
  mean = _mean_from_acc(acc_ref, _W1, _D1)
  h = (jnp.dot(x_ref[...], wr_ref[...], preferred_element_type=jnp.float32)
       + jnp.dot(mean, wn_ref[...], preferred_element_type=jnp.float32)
       + b_ref[...])
  t2_ref[...] = _split_tables(_bn_relu(h, g_ref[...], be_ref[...]), _W2, _H1)


def _dense2_body(t2_ref, acc_ref, wr_ref, wn_ref, b_ref, g_ref, be_ref,
                 w3n_ref, h2_ref, t3_ref):
  h1 = jnp.concatenate([t2_ref[0, :, :], t2_ref[1, :, :_H1 - _W2]], axis=1)
  mean = _mean_from_acc(acc_ref, _W2, _H1)
  h = (jnp.dot(h1, wr_ref[...], preferred_element_type=jnp.float32)
       + jnp.dot(mean, wn_ref[...], preferred_element_type=jnp.float32)
       + b_ref[...])
  h2 = _bn_relu(h, g_ref[...], be_ref[...])
  h2_ref[...] = h2
  y3 = jnp.dot(h2, w3n_ref[...], preferred_element_type=jnp.float32)
  t3_ref[...] = _split_tables(y3, _W3, _H3)


def _dense3_body(h2_ref, acc_ref, wr_ref, b_ref, g_ref, be_ref, wl_ref,
                 bl_ref, out_ref):
  mean = _mean_from_acc(acc_ref, _W3, _H3)
  h = (jnp.dot(h2_ref[...], wr_ref[...], preferred_element_type=jnp.float32)
       + mean + b_ref[...])
  h3 = _bn_relu(h, g_ref[...], be_ref[...])
  out_ref[...] = (jnp.dot(h3, wl_ref[...], preferred_element_type=jnp.float32)
                  + bl_ref[...])


_dense1 = pl.pallas_call(
    _dense1_body, out_shape=jax.ShapeDtypeStruct((2, _N, _W2), jnp.float32))
_dense2 = pl.pallas_call(
    _dense2_body, out_shape=(jax.ShapeDtypeStruct((_N, _H2), jnp.float32),
                             jax.ShapeDtypeStruct((2, _N, _W3), jnp.float32)))
_dense3 = pl.pallas_call(
    _dense3_body, out_shape=jax.ShapeDtypeStruct((_N, _C), jnp.float32))


def _row(v):
  return v.reshape(1, -1)


def kernel(x, edge_index_rsr, edge_index_rtr, edge_index_rur, W1r, W1n, b1,
           g1, be1, W2r, W2n, b2, g2, be2, W3r, W3n, b3, g3, be3, Wl, bl):
  s1, d1 = _prep_edges(edge_index_rsr)
  s2, d2 = _prep_edges(edge_index_rtr)
  s3, d3 = _prep_edges(edge_index_rur)

  t1 = _split_tables(x, _W1, _D1)
  acc1 = _seg_sum(_W1)(t1, s1, d1, jnp.zeros((_NPAD, _W1), jnp.float32))
  t2 = _dense1(x, acc1, W1r, W1n, _row(b1), _row(g1), _row(be1))

  acc2 = _seg_sum(_W2)(t2, s2, d2, jnp.zeros((_NPAD, _W2), jnp.float32))
  h2, t3 = _dense2(t2, acc2, W2r, W2n, _row(b2), _row(g2), _row(be2), W3n)

  acc3 = _seg_sum(_W3)(t3, s3, d3, jnp.zeros((_NPAD, _W3), jnp.float32))
  return _dense3(h2, acc3, W3r, _row(b3), _row(g3), _row(be3), Wl, _row(bl))


# speedup vs baseline: 3.9633x; 1.1224x over previous
import functools

import jax
import jax.numpy as jnp
from jax import lax
from jax.experimental import pallas as pl
from jax.experimental.pallas import tpu as pltpu
from jax.experimental.pallas import tpu_sc as plsc

_N = 10000
_E = 320000
_LANES = 128
_NTILE = 16
_NCHUNK = 160
_EPT = _NCHUNK * _LANES
_EPAD = _EPT * _NTILE
_RPT = 632
_NPAD = _NTILE * _RPT

_D1, _H1, _H2, _H3, _C = 128, 160, 256, 64, 64
_W1, _W2, _W3 = 80, 96, 48


@functools.lru_cache(maxsize=None)
def _seg_sum(w):
  mesh = plsc.VectorSubcoreMesh(core_axis_name="c", subcore_axis_name="s")

  def body(tab2, src2d, dst2d, zeros, out, src_v, dst_v, rows_a, rows_b,
           acc, sem_a, sem_b):
    cid = lax.axis_index("c")
    sid = lax.axis_index("s")
    pltpu.sync_copy(zeros.at[pl.ds(sid * _RPT, _RPT)],
                    acc.at[pl.ds(sid * _RPT, _RPT)])
    pltpu.sync_copy(src2d.at[pl.ds(sid * _NCHUNK, _NCHUNK)], src_v)
    pltpu.sync_copy(dst2d.at[pl.ds(sid * _NCHUNK, _NCHUNK)], dst_v)
    plsc.subcore_barrier()
    tab = tab2.at[cid]

    pltpu.async_copy(tab.at[src_v.at[0]], rows_a, sem_a)

    @pl.loop(0, _NCHUNK // 2)
    def _(p):
      ja = 2 * p
      jb = ja + 1
      pltpu.make_async_copy(tab.at[src_v.at[ja]], rows_a, sem_a).wait()
      pltpu.async_copy(tab.at[src_v.at[jb]], rows_b, sem_b)
      pltpu.sync_copy(rows_a, acc.at[dst_v.at[ja]], add=True)
      pltpu.make_async_copy(tab.at[src_v.at[jb]], rows_b, sem_b).wait()

      @pl.when(jb + 1 < _NCHUNK)
      def _():
        pltpu.async_copy(tab.at[src_v.at[jb + 1]], rows_a, sem_a)

      pltpu.sync_copy(rows_b, acc.at[dst_v.at[jb]], add=True)

    plsc.subcore_barrier()
    pltpu.sync_copy(acc.at[pl.ds(sid * _RPT, _RPT)],
                    out.at[cid, pl.ds(sid * _RPT, _RPT)])

  return pl.kernel(
      body,
      out_type=jax.ShapeDtypeStruct((2, _NPAD, w), jnp.float32),
      mesh=mesh,
      compiler_params=pltpu.CompilerParams(use_tc_tiling_on_sc=False),
      scratch_types=[
          pltpu.VMEM((_NCHUNK, _LANES), jnp.int32),
          pltpu.VMEM((_NCHUNK, _LANES), jnp.int32),
          pltpu.VMEM((_LANES, w), jnp.float32),
          pltpu.VMEM((_LANES, w), jnp.float32),
          pltpu.VMEM_SHARED((_NPAD, w), jnp.float32),
          pltpu.SemaphoreType.DMA,
          pltpu.SemaphoreType.DMA,
      ],
  )


def _prep_edges(ei):
  pad = _EPAD - _E
  src = jnp.concatenate([ei[0], jnp.zeros((pad,), jnp.int32)])
  dst = jnp.concatenate([ei[1], jnp.full((pad,), _N, jnp.int32)])
  return src.reshape(-1, _LANES), dst.reshape(-1, _LANES)


def _mean_from_acc(acc_ref, w, d):
  aL = acc_ref[0, pl.ds(0, _N), :]
  aR = acc_ref[1, pl.ds(0, _N), :]
  agg = jnp.concatenate([aL, aR[:, :d - w]], axis=1)
  cnt = jnp.maximum(aR[:, d - w:d - w + 1], 1.0)
  return agg / cnt


def _bn_relu(h, g, be):
  h = jnp.maximum(h, 0.0)
  m = jnp.mean(h, axis=0, keepdims=True)
  v = jnp.mean((h - m) ** 2, axis=0, keepdims=True)
  return (h - m) * lax.rsqrt(v + 1e-5) * g + be


def _split_tables(h, w, d):
  rpad = 2 * w - d - 1
  right = jnp.concatenate(
      [h[:, w:d], jnp.full((_N, 1), 1.0, jnp.float32),
       jnp.zeros((_N, rpad), jnp.float32)], axis=1)
  return jnp.stack([h[:, :w], right])


def _dense1_body(x_ref, acc_ref, wr_ref, wn_ref, b_ref, g_ref, be_ref,
                 t2_ref):
  mean = _mean_from_acc(acc_ref, _W1, _D1)
  h = (jnp.dot(x_ref[...], wr_ref[...], preferred_element_type=jnp.float32)
       + jnp.dot(mean, wn_ref[...], preferred_element_type=jnp.float32)
       + b_ref[...])
  t2_ref[...] = _split_tables(_bn_relu(h, g_ref[...], be_ref[...]), _W2, _H1)


def _dense2_body(t2_ref, acc_ref, wr_ref, wn_ref, b_ref, g_ref, be_ref,
                 w3n_ref, h2_ref, t3_ref):
  h1 = jnp.concatenate([t2_ref[0, :, :], t2_ref[1, :, :_H1 - _W2]], axis=1)
  mean = _mean_from_acc(acc_ref, _W2, _H1)
  h = (jnp.dot(h1, wr_ref[...], preferred_element_type=jnp.float32)
       + jnp.dot(mean, wn_ref[...], preferred_element_type=jnp.float32)
       + b_ref[...])
  h2 = _bn_relu(h, g_ref[...], be_ref[...])
  h2_ref[...] = h2
  y3 = jnp.dot(h2, w3n_ref[...], preferred_element_type=jnp.float32)
  t3_ref[...] = _split_tables(y3, _W3, _H3)


def _dense3_body(h2_ref, acc_ref, wr_ref, b_ref, g_ref, be_ref, wl_ref,
                 bl_ref, out_ref):
  mean = _mean_from_acc(acc_ref, _W3, _H3)
  h = (jnp.dot(h2_ref[...], wr_ref[...], preferred_element_type=jnp.float32)
       + mean + b_ref[...])
  h3 = _bn_relu(h, g_ref[...], be_ref[...])
  out_ref[...] = (jnp.dot(h3, wl_ref[...], preferred_element_type=jnp.float32)
                  + bl_ref[...])


_dense1 = pl.pallas_call(
    _dense1_body, out_shape=jax.ShapeDtypeStruct((2, _N, _W2), jnp.float32))
_dense2 = pl.pallas_call(
    _dense2_body, out_shape=(jax.ShapeDtypeStruct((_N, _H2), jnp.float32),
                             jax.ShapeDtypeStruct((2, _N, _W3), jnp.float32)))
_dense3 = pl.pallas_call(
    _dense3_body, out_shape=jax.ShapeDtypeStruct((_N, _C), jnp.float32))


def _row(v):
  return v.reshape(1, -1)


def kernel(x, edge_index_rsr, edge_index_rtr, edge_index_rur, W1r, W1n, b1,
           g1, be1, W2r, W2n, b2, g2, be2, W3r, W3n, b3, g3, be3, Wl, bl):
  s1, d1 = _prep_edges(edge_index_rsr)
  s2, d2 = _prep_edges(edge_index_rtr)
  s3, d3 = _prep_edges(edge_index_rur)

  t1 = _split_tables(x, _W1, _D1)
  acc1 = _seg_sum(_W1)(t1, s1, d1, jnp.zeros((_NPAD, _W1), jnp.float32))
  t2 = _dense1(x, acc1, W1r, W1n, _row(b1), _row(g1), _row(be1))

  acc2 = _seg_sum(_W2)(t2, s2, d2, jnp.zeros((_NPAD, _W2), jnp.float32))
  h2, t3 = _dense2(t2, acc2, W2r, W2n, _row(b2), _row(g2), _row(be2), W3n)

  acc3 = _seg_sum(_W3)(t3, s3, d3, jnp.zeros((_NPAD, _W3), jnp.float32))
  return _dense3(h2, acc3, W3r, _row(b3), _row(g3), _row(be3), Wl, _row(bl))


# speedup vs baseline: 4.4889x; 1.1326x over previous
import functools

import jax
import jax.numpy as jnp
from jax import lax
from jax.experimental import pallas as pl
from jax.experimental.pallas import tpu as pltpu
from jax.experimental.pallas import tpu_sc as plsc

_N = 10000
_E = 320000
_LANES = 128
_NTILE = 16
_NCHUNK = 160
_EPT = _NCHUNK * _LANES
_EPAD = _EPT * _NTILE
_NBUF = 4
_IBLK = 8
_RPT = 632
_NPAD = _NTILE * _RPT

_D1, _H1, _H2, _H3, _C = 128, 160, 256, 64, 64
_W1, _W2, _W3 = 80, 96, 48


@functools.lru_cache(maxsize=None)
def _seg_sum(w):
  mesh = plsc.VectorSubcoreMesh(core_axis_name="c", subcore_axis_name="s")

  def body(tab2, src2d, dst2d, zeros, out, src_i, dst_i, rows, acc,
           gsems, ssems, isem):
    cid = lax.axis_index("c")
    sid = lax.axis_index("s")
    pltpu.sync_copy(zeros.at[pl.ds(sid * _RPT, _RPT)],
                    acc.at[pl.ds(sid * _RPT, _RPT)])
    ibase = sid * _NCHUNK
    pltpu.sync_copy(src2d.at[pl.ds(ibase, 2 * _IBLK)],
                    src_i.at[pl.ds(0, 2 * _IBLK)])
    pltpu.sync_copy(dst2d.at[pl.ds(ibase, 2 * _IBLK)],
                    dst_i.at[pl.ds(0, 2 * _IBLK)])
    pltpu.async_copy(src2d.at[pl.ds(ibase + 2 * _IBLK, _IBLK)],
                     src_i.at[pl.ds(2 * _IBLK, _IBLK)], isem)
    pltpu.async_copy(dst2d.at[pl.ds(ibase + 2 * _IBLK, _IBLK)],
                     dst_i.at[pl.ds(2 * _IBLK, _IBLK)], isem)
    plsc.subcore_barrier()
    tab = tab2.at[cid]

    def idx_row(c):
      return lax.rem(lax.div(c, _IBLK), 3) * _IBLK + lax.rem(c, _IBLK)

    for b in range(_NBUF):
      pltpu.async_copy(tab.at[src_i.at[b]], rows[b], gsems.at[b])

    @pl.loop(0, _NCHUNK // _NBUF)
    def _(g):
      base = g * _NBUF
      even = lax.rem(g, 2) == 0
      blk = lax.div(g, 2)

      @pl.when(jnp.logical_and(even, g >= 2))
      def _():
        pltpu.make_async_copy(src2d.at[pl.ds(ibase, _IBLK)],
                              src_i.at[pl.ds(0, _IBLK)], isem).wait()
        pltpu.make_async_copy(dst2d.at[pl.ds(ibase, _IBLK)],
                              dst_i.at[pl.ds(0, _IBLK)], isem).wait()

      @pl.when(jnp.logical_and(even, blk + 2 < _NCHUNK // _IBLK))
      def _():
        slot = lax.rem(blk + 2, 3) * _IBLK
        off = ibase + (blk + 2) * _IBLK
        pltpu.async_copy(src2d.at[pl.ds(off, _IBLK)],
                         src_i.at[pl.ds(slot, _IBLK)], isem)
        pltpu.async_copy(dst2d.at[pl.ds(off, _IBLK)],
                         dst_i.at[pl.ds(slot, _IBLK)], isem)

      descs = []
      for b in range(_NBUF):
        j = base + b
        pltpu.make_async_copy(tab.at[src_i.at[b]], rows[b],
                              gsems.at[b]).wait()
        descs.append(pltpu.async_copy(rows[b], acc.at[dst_i.at[idx_row(j)]],
                                      ssems.at[b], add=True))
      for b in range(_NBUF):
        descs[b].wait()
        c = base + _NBUF + b

        @pl.when(c < _NCHUNK)
        def _():
          pltpu.async_copy(tab.at[src_i.at[idx_row(c)]], rows[b],
                           gsems.at[b])

    plsc.subcore_barrier()
    pltpu.sync_copy(acc.at[pl.ds(sid * _RPT, _RPT)],
                    out.at[cid, pl.ds(sid * _RPT, _RPT)])

  return pl.kernel(
      body,
      out_type=jax.ShapeDtypeStruct((2, _NPAD, w), jnp.float32),
      mesh=mesh,
      compiler_params=pltpu.CompilerParams(use_tc_tiling_on_sc=False),
      scratch_types=[
          pltpu.VMEM((3 * _IBLK, _LANES), jnp.int32),
          pltpu.VMEM((3 * _IBLK, _LANES), jnp.int32),
          [pltpu.VMEM((_LANES, w), jnp.float32) for _ in range(_NBUF)],
          pltpu.VMEM_SHARED((_NPAD, w), jnp.float32),
          pltpu.SemaphoreType.DMA((_NBUF,)),
          pltpu.SemaphoreType.DMA((_NBUF,)),
          pltpu.SemaphoreType.DMA,
      ],
  )


def _prep_edges(ei):
  pad = _EPAD - _E
  src = jnp.concatenate([ei[0], jnp.zeros((pad,), jnp.int32)])
  dst = jnp.concatenate([ei[1], jnp.full((pad,), _N, jnp.int32)])
  return src.reshape(-1, _LANES), dst.reshape(-1, _LANES)


def _mean_from_acc(acc_ref, w, d):
  aL = acc_ref[0, pl.ds(0, _N), :]
  aR = acc_ref[1, pl.ds(0, _N), :]
  agg = jnp.concatenate([aL, aR[:, :d - w]], axis=1)
  cnt = jnp.maximum(aR[:, d - w:d - w + 1], 1.0)
  return agg / cnt


def _bn_relu(h, g, be):
  h = jnp.maximum(h, 0.0)
  m = jnp.mean(h, axis=0, keepdims=True)
  v = jnp.mean((h - m) ** 2, axis=0, keepdims=True)
  return (h - m) * lax.rsqrt(v + 1e-5) * g + be


def _split_tables(h, w, d):
  rpad = 2 * w - d - 1
  right = jnp.concatenate(
      [h[:, w:d], jnp.full((_N, 1), 1.0, jnp.float32),
       jnp.zeros((_N, rpad), jnp.float32)], axis=1)
  return jnp.stack([h[:, :w], right])


def _dense1_body(x_ref, acc_ref, wr_ref, wn_ref, b_ref, g_ref, be_ref,
                 t2_ref):
  mean = _mean_from_acc(acc_ref, _W1, _D1)
  h = (jnp.dot(x_ref[...], wr_ref[...], preferred_element_type=jnp.float32)
       + jnp.dot(mean, wn_ref[...], preferred_element_type=jnp.float32)
       + b_ref[...])
  t2_ref[...] = _split_tables(_bn_relu(h, g_ref[...], be_ref[...]), _W2, _H1)


def _dense2_body(t2_ref, acc_ref, wr_ref, wn_ref, b_ref, g_ref, be_ref,
                 w3n_ref, h2_ref, t3_ref):
  h1 = jnp.concatenate([t2_ref[0, :, :], t2_ref[1, :, :_H1 - _W2]], axis=1)
  mean = _mean_from_acc(acc_ref, _W2, _H1)
  h = (jnp.dot(h1, wr_ref[...], preferred_element_type=jnp.float32)
       + jnp.dot(mean, wn_ref[...], preferred_element_type=jnp.float32)
       + b_ref[...])
  h2 = _bn_relu(h, g_ref[...], be_ref[...])
  h2_ref[...] = h2
  y3 = jnp.dot(h2, w3n_ref[...], preferred_element_type=jnp.float32)
  t3_ref[...] = _split_tables(y3, _W3, _H3)


def _dense3_body(h2_ref, acc_ref, wr_ref, b_ref, g_ref, be_ref, wl_ref,
                 bl_ref, out_ref):
  mean = _mean_from_acc(acc_ref, _W3, _H3)
  h = (jnp.dot(h2_ref[...], wr_ref[...], preferred_element_type=jnp.float32)
       + mean + b_ref[...])
  h3 = _bn_relu(h, g_ref[...], be_ref[...])
  out_ref[...] = (jnp.dot(h3, wl_ref[...], preferred_element_type=jnp.float32)
                  + bl_ref[...])


_dense1 = pl.pallas_call(
    _dense1_body, out_shape=jax.ShapeDtypeStruct((2, _N, _W2), jnp.float32))
_dense2 = pl.pallas_call(
    _dense2_body, out_shape=(jax.ShapeDtypeStruct((_N, _H2), jnp.float32),
                             jax.ShapeDtypeStruct((2, _N, _W3), jnp.float32)))
_dense3 = pl.pallas_call(
    _dense3_body, out_shape=jax.ShapeDtypeStruct((_N, _C), jnp.float32))


def _row(v):
  return v.reshape(1, -1)


def kernel(x, edge_index_rsr, edge_index_rtr, edge_index_rur, W1r, W1n, b1,
           g1, be1, W2r, W2n, b2, g2, be2, W3r, W3n, b3, g3, be3, Wl, bl):
  s1, d1 = _prep_edges(edge_index_rsr)
  s2, d2 = _prep_edges(edge_index_rtr)
  s3, d3 = _prep_edges(edge_index_rur)

  t1 = _split_tables(x, _W1, _D1)
  acc1 = _seg_sum(_W1)(t1, s1, d1, jnp.zeros((_NPAD, _W1), jnp.float32))
  t2 = _dense1(x, acc1, W1r, W1n, _row(b1), _row(g1), _row(be1))

  acc2 = _seg_sum(_W2)(t2, s2, d2, jnp.zeros((_NPAD, _W2), jnp.float32))
  h2, t3 = _dense2(t2, acc2, W2r, W2n, _row(b2), _row(g2), _row(be2), W3n)

  acc3 = _seg_sum(_W3)(t3, s3, d3, jnp.zeros((_NPAD, _W3), jnp.float32))
  return _dense3(h2, acc3, W3r, _row(b3), _row(g3), _row(be3), Wl, _row(bl))


# speedup vs baseline: 5.4276x; 1.2091x over previous
import functools

import jax
import jax.numpy as jnp
from jax import lax
from jax.experimental import pallas as pl
from jax.experimental.pallas import tpu as pltpu
from jax.experimental.pallas import tpu_sc as plsc

_N = 10000
_E = 320000
_LANES = 128
_NTILE = 16
_NCHUNK = 160
_EPT = _NCHUNK * _LANES
_EPAD = _EPT * _NTILE
_NBUF = 4
_IBLK = 8
_RPT = 632
_NPAD = _NTILE * _RPT
_HR = 80

_D1, _H1, _H2, _H3, _C = 128, 160, 256, 64, 64
_W1, _W2, _W3 = _D1 // 2, _H1 // 2, _H3 // 2


@functools.lru_cache(maxsize=None)
def _seg_sum(w):
  mesh = plsc.VectorSubcoreMesh(core_axis_name="c", subcore_axis_name="s")

  def body(tab2, src2d, dst2d, zeros, out, out_cnt, src_i, dst_i, rows,
           hist, iidx, acc, cnt_sh, gsems, ssems, isem):
    cid = lax.axis_index("c")
    sid = lax.axis_index("s")
    zero16 = jnp.zeros((16,), jnp.float32)
    one16 = jnp.ones((16,), jnp.float32)
    iota16 = lax.iota(jnp.int32, 16)

    pltpu.sync_copy(zeros.at[pl.ds(sid * _RPT, _RPT)],
                    acc.at[pl.ds(sid * _RPT, _RPT)])

    @pl.loop(0, _HR)
    def _(r):
      for k in range(8):
        hist[r, pl.ds(16 * k, 16)] = zero16

    @pl.loop(0, _HR // 16)
    def _(r):
      iidx[pl.ds(16 * r, 16)] = iota16 + 16 * r

    @pl.when(sid < 10)
    def _():
      pltpu.sync_copy(hist.at[pl.ds(0, 8)], cnt_sh.at[pl.ds(sid * 8, 8)])

    ibase = sid * _NCHUNK
    pltpu.sync_copy(src2d.at[pl.ds(ibase, 2 * _IBLK)],
                    src_i.at[pl.ds(0, 2 * _IBLK)])
    pltpu.sync_copy(dst2d.at[pl.ds(ibase, 2 * _IBLK)],
                    dst_i.at[pl.ds(0, 2 * _IBLK)])
    pltpu.async_copy(src2d.at[pl.ds(ibase + 2 * _IBLK, _IBLK)],
                     src_i.at[pl.ds(2 * _IBLK, _IBLK)], isem)
    pltpu.async_copy(dst2d.at[pl.ds(ibase + 2 * _IBLK, _IBLK)],
                     dst_i.at[pl.ds(2 * _IBLK, _IBLK)], isem)
    plsc.subcore_barrier()
    tab = tab2.at[cid]

    def idx_row(c):
      return lax.rem(lax.div(c, _IBLK), 3) * _IBLK + lax.rem(c, _IBLK)

    for b in range(_NBUF):
      pltpu.async_copy(tab.at[src_i.at[b]], rows[b], gsems.at[b])

    @pl.loop(0, _NCHUNK // _NBUF)
    def _(g):
      base = g * _NBUF
      even = lax.rem(g, 2) == 0
      blk = lax.div(g, 2)

      @pl.when(jnp.logical_and(even, g >= 2))
      def _():
        pltpu.make_async_copy(src2d.at[pl.ds(ibase, _IBLK)],
                              src_i.at[pl.ds(0, _IBLK)], isem).wait()
        pltpu.make_async_copy(dst2d.at[pl.ds(ibase, _IBLK)],
                              dst_i.at[pl.ds(0, _IBLK)], isem).wait()

      @pl.when(jnp.logical_and(even, blk + 2 < _NCHUNK // _IBLK))
      def _():
        slot = lax.rem(blk + 2, 3) * _IBLK
        off = ibase + (blk + 2) * _IBLK
        pltpu.async_copy(src2d.at[pl.ds(off, _IBLK)],
                         src_i.at[pl.ds(slot, _IBLK)], isem)
        pltpu.async_copy(dst2d.at[pl.ds(off, _IBLK)],
                         dst_i.at[pl.ds(slot, _IBLK)], isem)

      descs = []
      for b in range(_NBUF):
        j = base + b
        pltpu.make_async_copy(tab.at[src_i.at[b]], rows[b],
                              gsems.at[b]).wait()
        descs.append(pltpu.async_copy(rows[b], acc.at[dst_i.at[idx_row(j)]],
                                      ssems.at[b], add=True))

      for b in range(_NBUF):
        r = idx_row(base + b)
        for k in range(8):
          v = dst_i[r, pl.ds(16 * k, 16)]
          plsc.addupdate_scatter(
              hist, [lax.shift_right_logical(v, 7),
                     jnp.bitwise_and(v, 127)], one16)

      for b in range(_NBUF):
        descs[b].wait()
        c = base + _NBUF + b

        @pl.when(c < _NCHUNK)
        def _():
          pltpu.async_copy(tab.at[src_i.at[idx_row(c)]], rows[b],
                           gsems.at[b])

    pltpu.sync_copy(hist, cnt_sh.at[iidx], add=True)
    plsc.subcore_barrier()
    pltpu.sync_copy(acc.at[pl.ds(sid * _RPT, _RPT)],
                    out.at[cid, pl.ds(sid * _RPT, _RPT)])

    @pl.when(sid < 10)
    def _():
      pltpu.sync_copy(cnt_sh.at[pl.ds(sid * 8, 8)],
                      out_cnt.at[cid, pl.ds(sid * 8, 8)])

  return pl.kernel(
      body,
      out_type=(jax.ShapeDtypeStruct((2, _NPAD, w), jnp.float32),
                jax.ShapeDtypeStruct((2, _HR, 128), jnp.float32)),
      mesh=mesh,
      compiler_params=pltpu.CompilerParams(use_tc_tiling_on_sc=False,
                                           needs_layout_passes=False),
      scratch_types=[
          pltpu.VMEM((3 * _IBLK, _LANES), jnp.int32),
          pltpu.VMEM((3 * _IBLK, _LANES), jnp.int32),
          [pltpu.VMEM((_LANES, w), jnp.float32) for _ in range(_NBUF)],
          pltpu.VMEM((_HR, 128), jnp.float32),
          pltpu.VMEM((_HR,), jnp.int32),
          pltpu.VMEM_SHARED((_NPAD, w), jnp.float32),
          pltpu.VMEM_SHARED((_HR, 128), jnp.float32),
          pltpu.SemaphoreType.DMA((_NBUF,)),
          pltpu.SemaphoreType.DMA((_NBUF,)),
          pltpu.SemaphoreType.DMA,
      ],
  )


def _prep_edges(ei):
  pad = _EPAD - _E
  src = jnp.concatenate([ei[0], jnp.zeros((pad,), jnp.int32)])
  dst = jnp.concatenate([ei[1], jnp.full((pad,), _N, jnp.int32)])
  return src.reshape(-1, _LANES), dst.reshape(-1, _LANES)


def _mean_from_acc(acc_ref, cnt_ref):
  agg = jnp.concatenate([acc_ref[0, pl.ds(0, _N), :],
                         acc_ref[1, pl.ds(0, _N), :]], axis=1)
  return agg / jnp.maximum(cnt_ref[...], 1.0)


def _bn_relu(h, g, be):
  h = jnp.maximum(h, 0.0)
  m = jnp.mean(h, axis=0, keepdims=True)
  v = jnp.mean((h - m) ** 2, axis=0, keepdims=True)
  return (h - m) * lax.rsqrt(v + 1e-5) * g + be


def _split_tables(h, w):
  return jnp.stack([h[:, :w], h[:, w:]])


def _dense1_body(x_ref, acc_ref, cnt_ref, wr_ref, wn_ref, b_ref, g_ref,
                 be_ref, t2_ref):
  mean = _mean_from_acc(acc_ref, cnt_ref)
  h = (jnp.dot(x_ref[...], wr_ref[...], preferred_element_type=jnp.float32)
       + jnp.dot(mean, wn_ref[...], preferred_element_type=jnp.float32)
       + b_ref[...])
  t2_ref[...] = _split_tables(_bn_relu(h, g_ref[...], be_ref[...]), _W2)


def _dense2_body(t2_ref, acc_ref, cnt_ref, wr_ref, wn_ref, b_ref, g_ref,
                 be_ref, w3n_ref, h2_ref, t3_ref):
  h1 = jnp.concatenate([t2_ref[0, :, :], t2_ref[1, :, :]], axis=1)
  mean = _mean_from_acc(acc_ref, cnt_ref)
  h = (jnp.dot(h1, wr_ref[...], preferred_element_type=jnp.float32)
       + jnp.dot(mean, wn_ref[...], preferred_element_type=jnp.float32)
       + b_ref[...])
  h2 = _bn_relu(h, g_ref[...], be_ref[...])
  h2_ref[...] = h2
  y3 = jnp.dot(h2, w3n_ref[...], preferred_element_type=jnp.float32)
  t3_ref[...] = _split_tables(y3, _W3)


def _dense3_body(h2_ref, acc_ref, cnt_ref, wr_ref, b_ref, g_ref, be_ref,
                 wl_ref, bl_ref, out_ref):
  mean = _mean_from_acc(acc_ref, cnt_ref)
  h = (jnp.dot(h2_ref[...], wr_ref[...], preferred_element_type=jnp.float32)
       + mean + b_ref[...])
  h3 = _bn_relu(h, g_ref[...], be_ref[...])
  out_ref[...] = (jnp.dot(h3, wl_ref[...], preferred_element_type=jnp.float32)
                  + bl_ref[...])


_dense1 = pl.pallas_call(
    _dense1_body, out_shape=jax.ShapeDtypeStruct((2, _N, _W2), jnp.float32))
_dense2 = pl.pallas_call(
    _dense2_body, out_shape=(jax.ShapeDtypeStruct((_N, _H2), jnp.float32),
                             jax.ShapeDtypeStruct((2, _N, _W3), jnp.float32)))
_dense3 = pl.pallas_call(
    _dense3_body, out_shape=jax.ShapeDtypeStruct((_N, _C), jnp.float32))


def _row(v):
  return v.reshape(1, -1)


def _cnt_col(out_cnt):
  return out_cnt[0].reshape(-1)[:_N].reshape(_N, 1)


def kernel(x, edge_index_rsr, edge_index_rtr, edge_index_rur, W1r, W1n, b1,
           g1, be1, W2r, W2n, b2, g2, be2, W3r, W3n, b3, g3, be3, Wl, bl):
  s1, d1 = _prep_edges(edge_index_rsr)
  s2, d2 = _prep_edges(edge_index_rtr)
  s3, d3 = _prep_edges(edge_index_rur)

  t1 = _split_tables(x, _W1)
  acc1, cnt1 = _seg_sum(_W1)(t1, s1, d1,
                             jnp.zeros((_NPAD, _W1), jnp.float32))
  t2 = _dense1(x, acc1, _cnt_col(cnt1), W1r, W1n, _row(b1), _row(g1),
               _row(be1))

  acc2, cnt2 = _seg_sum(_W2)(t2, s2, d2,
                             jnp.zeros((_NPAD, _W2), jnp.float32))
  h2, t3 = _dense2(t2, acc2, _cnt_col(cnt2), W2r, W2n, _row(b2), _row(g2),
                   _row(be2), W3n)

  acc3, cnt3 = _seg_sum(_W3)(t3, s3, d3,
                             jnp.zeros((_NPAD, _W3), jnp.float32))
  return _dense3(h2, acc3, _cnt_col(cnt3), W3r, _row(b3), _row(g3),
                 _row(be3), Wl, _row(bl))


# speedup vs baseline: 5.4536x; 1.0048x over previous
import functools

import jax
import jax.numpy as jnp
from jax import lax
from jax.experimental import pallas as pl
from jax.experimental.pallas import tpu as pltpu
from jax.experimental.pallas import tpu_sc as plsc

_N = 10000
_E = 320000
_LANES = 128
_NTILE = 16
_NCHUNK = 160
_EPT = _NCHUNK * _LANES
_EPAD = _EPT * _NTILE
_NBUF = 4
_IBLK = 8
_RPT = 632
_NPAD = _NTILE * _RPT
_HR = 80

_D1, _H1, _H2, _H3, _C = 128, 160, 256, 64, 64
_W1, _W2, _W3 = _D1 // 2, _H1 // 2, _H3 // 2


@functools.lru_cache(maxsize=None)
def _seg_sum(w):
  mesh = plsc.VectorSubcoreMesh(core_axis_name="c", subcore_axis_name="s")
  nbuf = _NBUF

  def body(tab2, src2d, dst2d, zeros, out, out_cnt, src_i, dst_i, rows,
           hist, iidx, acc, cnt_sh, gsems, ssems, isem):
    cid = lax.axis_index("c")
    sid = lax.axis_index("s")
    zero16 = jnp.zeros((16,), jnp.float32)
    one16 = jnp.ones((16,), jnp.float32)
    iota16 = lax.iota(jnp.int32, 16)

    pltpu.sync_copy(zeros.at[pl.ds(sid * _RPT, _RPT)],
                    acc.at[pl.ds(sid * _RPT, _RPT)])

    @pl.loop(0, _HR)
    def _(r):
      for k in range(8):
        hist[r, pl.ds(16 * k, 16)] = zero16

    @pl.loop(0, _HR // 16)
    def _(r):
      iidx[pl.ds(16 * r, 16)] = iota16 + 16 * r

    @pl.when(sid < 10)
    def _():
      pltpu.sync_copy(hist.at[pl.ds(0, 8)], cnt_sh.at[pl.ds(sid * 8, 8)])

    ibase = sid * _NCHUNK
    pltpu.sync_copy(src2d.at[pl.ds(ibase, 2 * _IBLK)],
                    src_i.at[pl.ds(0, 2 * _IBLK)])
    pltpu.sync_copy(dst2d.at[pl.ds(ibase, 2 * _IBLK)],
                    dst_i.at[pl.ds(0, 2 * _IBLK)])
    if nbuf != _IBLK:
      pltpu.async_copy(src2d.at[pl.ds(ibase + 2 * _IBLK, _IBLK)],
                       src_i.at[pl.ds(2 * _IBLK, _IBLK)], isem)
      pltpu.async_copy(dst2d.at[pl.ds(ibase + 2 * _IBLK, _IBLK)],
                       dst_i.at[pl.ds(2 * _IBLK, _IBLK)], isem)
    plsc.subcore_barrier()
    tab = tab2.at[cid]

    def idx_row(c):
      return lax.rem(lax.div(c, _IBLK), 3) * _IBLK + lax.rem(c, _IBLK)

    for b in range(nbuf):
      pltpu.async_copy(tab.at[src_i.at[b]], rows[b], gsems.at[b])

    @pl.loop(0, _NCHUNK // nbuf)
    def _(g):
      base = g * nbuf
      if nbuf == _IBLK:
        do_drain = g >= 1
        do_pref = g + 2 < _NCHUNK // _IBLK
        pref_blk = g + 2
      else:
        even = lax.rem(g, 2) == 0
        blk = lax.div(g, 2)
        do_drain = jnp.logical_and(even, g >= 2)
        do_pref = jnp.logical_and(even, blk + 2 < _NCHUNK // _IBLK)
        pref_blk = blk + 2

      @pl.when(do_drain)
      def _():
        pltpu.make_async_copy(src2d.at[pl.ds(ibase, _IBLK)],
                              src_i.at[pl.ds(0, _IBLK)], isem).wait()
        pltpu.make_async_copy(dst2d.at[pl.ds(ibase, _IBLK)],
                              dst_i.at[pl.ds(0, _IBLK)], isem).wait()

      @pl.when(do_pref)
      def _():
        slot = lax.rem(pref_blk, 3) * _IBLK
        off = ibase + pref_blk * _IBLK
        pltpu.async_copy(src2d.at[pl.ds(off, _IBLK)],
                         src_i.at[pl.ds(slot, _IBLK)], isem)
        pltpu.async_copy(dst2d.at[pl.ds(off, _IBLK)],
                         dst_i.at[pl.ds(slot, _IBLK)], isem)

      descs = []
      for b in range(nbuf):
        j = base + b
        pltpu.make_async_copy(tab.at[src_i.at[b]], rows[b],
                              gsems.at[b]).wait()
        descs.append(pltpu.async_copy(rows[b], acc.at[dst_i.at[idx_row(j)]],
                                      ssems.at[b], add=True))

      for b in range(nbuf):
        r = idx_row(base + b)
        for k in range(8):
          v = dst_i[r, pl.ds(16 * k, 16)]
          plsc.addupdate_scatter(
              hist, [lax.shift_right_logical(v, 7),
                     jnp.bitwise_and(v, 127)], one16)

      for b in range(nbuf):
        descs[b].wait()
        c = base + nbuf + b

        @pl.when(c < _NCHUNK)
        def _():
          pltpu.async_copy(tab.at[src_i.at[idx_row(c)]], rows[b],
                           gsems.at[b])

    pltpu.sync_copy(hist, cnt_sh.at[iidx], add=True)
    plsc.subcore_barrier()
    pltpu.sync_copy(acc.at[pl.ds(sid * _RPT, _RPT)],
                    out.at[cid, pl.ds(sid * _RPT, _RPT)])

    @pl.when(sid < 10)
    def _():
      pltpu.sync_copy(cnt_sh.at[pl.ds(sid * 8, 8)],
                      out_cnt.at[cid, pl.ds(sid * 8, 8)])

  return pl.kernel(
      body,
      out_type=(jax.ShapeDtypeStruct((2, _NPAD, w), jnp.float32),
                jax.ShapeDtypeStruct((2, _HR, 128), jnp.float32)),
      mesh=mesh,
      compiler_params=pltpu.CompilerParams(use_tc_tiling_on_sc=False,
                                           needs_layout_passes=False),
      scratch_types=[
          pltpu.VMEM((3 * _IBLK, _LANES), jnp.int32),
          pltpu.VMEM((3 * _IBLK, _LANES), jnp.int32),
          [pltpu.VMEM((_LANES, w), jnp.float32) for _ in range(nbuf)],
          pltpu.VMEM((_HR, 128), jnp.float32),
          pltpu.VMEM((_HR,), jnp.int32),
          pltpu.VMEM_SHARED((_NPAD, w), jnp.float32),
          pltpu.VMEM_SHARED((_HR, 128), jnp.float32),
          pltpu.SemaphoreType.DMA((nbuf,)),
          pltpu.SemaphoreType.DMA((nbuf,)),
          pltpu.SemaphoreType.DMA,
      ],
  )


def _prep_edges(ei):
  pad = _EPAD - _E
  src = jnp.concatenate([ei[0], jnp.zeros((pad,), jnp.int32)])
  dst = jnp.concatenate([ei[1], jnp.full((pad,), _N, jnp.int32)])
  return src.reshape(-1, _LANES), dst.reshape(-1, _LANES)


def _mean_from_acc(acc_ref, cnt_ref):
  agg = jnp.concatenate([acc_ref[0, pl.ds(0, _N), :],
                         acc_ref[1, pl.ds(0, _N), :]], axis=1)
  return agg / jnp.maximum(cnt_ref[...], 1.0)


def _bn_relu(h, g, be):
  h = jnp.maximum(h, 0.0)
  m = jnp.mean(h, axis=0, keepdims=True)
  v = jnp.mean((h - m) ** 2, axis=0, keepdims=True)
  return (h - m) * lax.rsqrt(v + 1e-5) * g + be


def _split_tables(h, w):
  return jnp.stack([h[:, :w], h[:, w:]])


def _dense1_body(x_ref, acc_ref, cnt_ref, wr_ref, wn_ref, b_ref, g_ref,
                 be_ref, t2_ref):
  mean = _mean_from_acc(acc_ref, cnt_ref)
  h = (jnp.dot(x_ref[...], wr_ref[...], preferred_element_type=jnp.float32)
       + jnp.dot(mean, wn_ref[...], preferred_element_type=jnp.float32)
       + b_ref[...])
  t2_ref[...] = _split_tables(_bn_relu(h, g_ref[...], be_ref[...]), _W2)


def _dense2_body(t2_ref, acc_ref, cnt_ref, wr_ref, wn_ref, b_ref, g_ref,
                 be_ref, w3n_ref, h2_ref, t3_ref):
  h1 = jnp.concatenate([t2_ref[0, :, :], t2_ref[1, :, :]], axis=1)
  mean = _mean_from_acc(acc_ref, cnt_ref)
  h = (jnp.dot(h1, wr_ref[...], preferred_element_type=jnp.float32)
       + jnp.dot(mean, wn_ref[...], preferred_element_type=jnp.float32)
       + b_ref[...])
  h2 = _bn_relu(h, g_ref[...], be_ref[...])
  h2_ref[...] = h2
  y3 = jnp.dot(h2, w3n_ref[...], preferred_element_type=jnp.float32)
  t3_ref[...] = _split_tables(y3, _W3)


def _dense3_body(h2_ref, acc_ref, cnt_ref, wr_ref, b_ref, g_ref, be_ref,
                 wl_ref, bl_ref, out_ref):
  mean = _mean_from_acc(acc_ref, cnt_ref)
  h = (jnp.dot(h2_ref[...], wr_ref[...], preferred_element_type=jnp.float32)
       + mean + b_ref[...])
  h3 = _bn_relu(h, g_ref[...], be_ref[...])
  out_ref[...] = (jnp.dot(h3, wl_ref[...], preferred_element_type=jnp.float32)
                  + bl_ref[...])


_dense1 = pl.pallas_call(
    _dense1_body, out_shape=jax.ShapeDtypeStruct((2, _N, _W2), jnp.float32))
_dense2 = pl.pallas_call(
    _dense2_body, out_shape=(jax.ShapeDtypeStruct((_N, _H2), jnp.float32),
                             jax.ShapeDtypeStruct((2, _N, _W3), jnp.float32)))
_dense3 = pl.pallas_call(
    _dense3_body, out_shape=jax.ShapeDtypeStruct((_N, _C), jnp.float32))


def _row(v):
  return v.reshape(1, -1)


def _cnt_col(out_cnt):
  return out_cnt[0].reshape(-1)[:_N].reshape(_N, 1)


def kernel(x, edge_index_rsr, edge_index_rtr, edge_index_rur, W1r, W1n, b1,
           g1, be1, W2r, W2n, b2, g2, be2, W3r, W3n, b3, g3, be3, Wl, bl):
  s1, d1 = _prep_edges(edge_index_rsr)
  s2, d2 = _prep_edges(edge_index_rtr)
  s3, d3 = _prep_edges(edge_index_rur)

  t1 = _split_tables(x, _W1)
  acc1, cnt1 = _seg_sum(_W1)(t1, s1, d1,
                             jnp.zeros((_NPAD, _W1), jnp.float32))
  t2 = _dense1(x, acc1, _cnt_col(cnt1), W1r, W1n, _row(b1), _row(g1),
               _row(be1))

  acc2, cnt2 = _seg_sum(_W2)(t2, s2, d2,
                             jnp.zeros((_NPAD, _W2), jnp.float32))
  h2, t3 = _dense2(t2, acc2, _cnt_col(cnt2), W2r, W2n, _row(b2), _row(g2),
                   _row(be2), W3n)

  acc3, cnt3 = _seg_sum(_W3)(t3, s3, d3,
                             jnp.zeros((_NPAD, _W3), jnp.float32))
  return _dense3(h2, acc3, _cnt_col(cnt3), W3r, _row(b3), _row(g3),
                 _row(be3), Wl, _row(bl))


# speedup vs baseline: 5.4546x; 1.0002x over previous
import functools

import jax
import jax.numpy as jnp
from jax import lax
from jax.experimental import pallas as pl
from jax.experimental.pallas import tpu as pltpu
from jax.experimental.pallas import tpu_sc as plsc

_N = 10000
_E = 320000
_LANES = 128
_NTILE = 16
_NCHUNK = 160
_EPT = _NCHUNK * _LANES
_EPAD = _EPT * _NTILE
_NBUF = 4
_IBLK = 8
_RPT = 632
_NPAD = _NTILE * _RPT
_HR = 80

_D1, _H1, _H2, _H3, _C = 128, 160, 256, 64, 64
_W1, _W2, _W3 = _D1 // 2, _H1 // 2, _H3 // 2


@functools.lru_cache(maxsize=None)
def _seg_sum(w):
  mesh = plsc.VectorSubcoreMesh(core_axis_name="c", subcore_axis_name="s")
  nbuf = _NBUF

  def body(tab2, src2d, dst2d, zeros, out, out_cnt, src_i, dst_i, rows,
           hist, iidx, acc, cnt_sh, gsems, ssems, isem):
    cid = lax.axis_index("c")
    sid = lax.axis_index("s")
    zero16 = jnp.zeros((16,), jnp.float32)
    one16 = jnp.ones((16,), jnp.float32)
    iota16 = lax.iota(jnp.int32, 16)

    pltpu.sync_copy(zeros.at[pl.ds(sid * _RPT, _RPT)],
                    acc.at[pl.ds(sid * _RPT, _RPT)])

    @pl.loop(0, _HR)
    def _(r):
      for k in range(8):
        hist[r, pl.ds(16 * k, 16)] = zero16

    @pl.loop(0, _HR // 16)
    def _(r):
      iidx[pl.ds(16 * r, 16)] = iota16 + 16 * r

    @pl.when(sid < 10)
    def _():
      pltpu.sync_copy(hist.at[pl.ds(0, 8)], cnt_sh.at[pl.ds(sid * 8, 8)])

    ibase = sid * _NCHUNK
    pltpu.sync_copy(src2d.at[pl.ds(ibase, 2 * _IBLK)],
                    src_i.at[pl.ds(0, 2 * _IBLK)])
    pltpu.sync_copy(dst2d.at[pl.ds(ibase, 2 * _IBLK)],
                    dst_i.at[pl.ds(0, 2 * _IBLK)])
    pltpu.async_copy(src2d.at[pl.ds(ibase + 2 * _IBLK, _IBLK)],
                     src_i.at[pl.ds(2 * _IBLK, _IBLK)], isem)
    pltpu.async_copy(dst2d.at[pl.ds(ibase + 2 * _IBLK, _IBLK)],
                     dst_i.at[pl.ds(2 * _IBLK, _IBLK)], isem)
    plsc.subcore_barrier()
    tab = tab2.at[cid]

    def idx_row(c):
      return lax.rem(lax.div(c, _IBLK), 3) * _IBLK + lax.rem(c, _IBLK)

    for b in range(nbuf):
      pltpu.async_copy(tab.at[src_i.at[b]], rows[b], gsems.at[b])

    @pl.loop(0, _NCHUNK // nbuf)
    def _(g):
      base = g * nbuf
      even = lax.rem(g, 2) == 0
      blk = lax.div(g, 2)
      do_drain = jnp.logical_and(even, g >= 2)
      do_pref = jnp.logical_and(even, blk + 2 < _NCHUNK // _IBLK)
      pref_blk = blk + 2

      @pl.when(do_drain)
      def _():
        pltpu.make_async_copy(src2d.at[pl.ds(ibase, _IBLK)],
                              src_i.at[pl.ds(0, _IBLK)], isem).wait()
        pltpu.make_async_copy(dst2d.at[pl.ds(ibase, _IBLK)],
                              dst_i.at[pl.ds(0, _IBLK)], isem).wait()

      @pl.when(do_pref)
      def _():
        slot = lax.rem(pref_blk, 3) * _IBLK
        off = ibase + pref_blk * _IBLK
        pltpu.async_copy(src2d.at[pl.ds(off, _IBLK)],
                         src_i.at[pl.ds(slot, _IBLK)], isem)
        pltpu.async_copy(dst2d.at[pl.ds(off, _IBLK)],
                         dst_i.at[pl.ds(slot, _IBLK)], isem)

      descs = []
      for b in range(nbuf):
        j = base + b
        pltpu.make_async_copy(tab.at[src_i.at[b]], rows[b],
                              gsems.at[b]).wait()
        descs.append(pltpu.async_copy(rows[b], acc.at[dst_i.at[idx_row(j)]],
                                      ssems.at[b], add=True))

      for b in range(nbuf):
        r = idx_row(base + b)
        for k in range(8):
          v = dst_i[r, pl.ds(16 * k, 16)]
          plsc.addupdate_scatter(
              hist, [lax.shift_right_logical(v, 7),
                     jnp.bitwise_and(v, 127)], one16)

      for b in range(nbuf):
        descs[b].wait()
        c = base + nbuf + b

        @pl.when(c < _NCHUNK)
        def _():
          pltpu.async_copy(tab.at[src_i.at[idx_row(c)]], rows[b],
                           gsems.at[b])

    pltpu.sync_copy(hist, cnt_sh.at[iidx], add=True)
    plsc.subcore_barrier()
    pltpu.sync_copy(acc.at[pl.ds(sid * _RPT, _RPT)],
                    out.at[cid, pl.ds(sid * _RPT, _RPT)])

    @pl.when(sid < 10)
    def _():
      pltpu.sync_copy(cnt_sh.at[pl.ds(sid * 8, 8)],
                      out_cnt.at[cid, pl.ds(sid * 8, 8)])

  return pl.kernel(
      body,
      out_type=(jax.ShapeDtypeStruct((2, _NPAD, w), jnp.float32),
                jax.ShapeDtypeStruct((2, _HR, 128), jnp.float32)),
      mesh=mesh,
      compiler_params=pltpu.CompilerParams(use_tc_tiling_on_sc=False,
                                           needs_layout_passes=False),
      scratch_types=[
          pltpu.VMEM((3 * _IBLK, _LANES), jnp.int32),
          pltpu.VMEM((3 * _IBLK, _LANES), jnp.int32),
          [pltpu.VMEM((_LANES, w), jnp.float32) for _ in range(nbuf)],
          pltpu.VMEM((_HR, 128), jnp.float32),
          pltpu.VMEM((_HR,), jnp.int32),
          pltpu.VMEM_SHARED((_NPAD, w), jnp.float32),
          pltpu.VMEM_SHARED((_HR, 128), jnp.float32),
          pltpu.SemaphoreType.DMA((nbuf,)),
          pltpu.SemaphoreType.DMA((nbuf,)),
          pltpu.SemaphoreType.DMA,
      ],
  )


def _prep_edges(ei):
  pad = _EPAD - _E
  src = jnp.concatenate([ei[0], jnp.zeros((pad,), jnp.int32)])
  dst = jnp.concatenate([ei[1], jnp.full((pad,), _N, jnp.int32)])
  return src.reshape(-1, _LANES), dst.reshape(-1, _LANES)


def _mean_from_acc(acc_ref, cnt_ref):
  agg = jnp.concatenate([acc_ref[0, pl.ds(0, _N), :],
                         acc_ref[1, pl.ds(0, _N), :]], axis=1)
  return agg / jnp.maximum(cnt_ref[...], 1.0)


def _bn_relu(h, g, be):
  h = jnp.maximum(h, 0.0)
  m = jnp.mean(h, axis=0, keepdims=True)
  v = jnp.mean((h - m) ** 2, axis=0, keepdims=True)
  return (h - m) * lax.rsqrt(v + 1e-5) * g + be


def _split_tables(h, w):
  return jnp.stack([h[:, :w], h[:, w:]])


def _dense1_body(x_ref, acc_ref, cnt_ref, wr_ref, wn_ref, b_ref, g_ref,
                 be_ref, t2_ref):
  mean = _mean_from_acc(acc_ref, cnt_ref)
  h = (jnp.dot(x_ref[...], wr_ref[...], preferred_element_type=jnp.float32)
       + jnp.dot(mean, wn_ref[...], preferred_element_type=jnp.float32)
       + b_ref[...])
  t2_ref[...] = _split_tables(_bn_relu(h, g_ref[...], be_ref[...]), _W2)


def _dense2_body(t2_ref, acc_ref, cnt_ref, wr_ref, wn_ref, b_ref, g_ref,
                 be_ref, w3n_ref, h2_ref, t3_ref):
  h1 = jnp.concatenate([t2_ref[0, :, :], t2_ref[1, :, :]], axis=1)
  mean = _mean_from_acc(acc_ref, cnt_ref)
  h = (jnp.dot(h1, wr_ref[...], preferred_element_type=jnp.float32)
       + jnp.dot(mean, wn_ref[...], preferred_element_type=jnp.float32)
       + b_ref[...])
  h2 = _bn_relu(h, g_ref[...], be_ref[...])
  h2_ref[...] = h2
  y3 = jnp.dot(h2, w3n_ref[...], preferred_element_type=jnp.float32)
  t3_ref[...] = _split_tables(y3, _W3)


def _dense3_body(h2_ref, acc_ref, cnt_ref, wr_ref, b_ref, g_ref, be_ref,
                 wl_ref, bl_ref, out_ref):
  mean = _mean_from_acc(acc_ref, cnt_ref)
  h = (jnp.dot(h2_ref[...], wr_ref[...], preferred_element_type=jnp.float32)
       + mean + b_ref[...])
  h3 = _bn_relu(h, g_ref[...], be_ref[...])
  out_ref[...] = (jnp.dot(h3, wl_ref[...], preferred_element_type=jnp.float32)
                  + bl_ref[...])


_dense1 = pl.pallas_call(
    _dense1_body, out_shape=jax.ShapeDtypeStruct((2, _N, _W2), jnp.float32))
_dense2 = pl.pallas_call(
    _dense2_body, out_shape=(jax.ShapeDtypeStruct((_N, _H2), jnp.float32),
                             jax.ShapeDtypeStruct((2, _N, _W3), jnp.float32)))
_dense3 = pl.pallas_call(
    _dense3_body, out_shape=jax.ShapeDtypeStruct((_N, _C), jnp.float32))


def _row(v):
  return v.reshape(1, -1)


def _cnt_col(out_cnt):
  return out_cnt[0].reshape(-1)[:_N].reshape(_N, 1)


def kernel(x, edge_index_rsr, edge_index_rtr, edge_index_rur, W1r, W1n, b1,
           g1, be1, W2r, W2n, b2, g2, be2, W3r, W3n, b3, g3, be3, Wl, bl):
  s1, d1 = _prep_edges(edge_index_rsr)
  s2, d2 = _prep_edges(edge_index_rtr)
  s3, d3 = _prep_edges(edge_index_rur)

  t1 = _split_tables(x, _W1)
  acc1, cnt1 = _seg_sum(_W1)(t1, s1, d1,
                             jnp.zeros((_NPAD, _W1), jnp.float32))
  t2 = _dense1(x, acc1, _cnt_col(cnt1), W1r, W1n, _row(b1), _row(g1),
               _row(be1))

  acc2, cnt2 = _seg_sum(_W2)(t2, s2, d2,
                             jnp.zeros((_NPAD, _W2), jnp.float32))
  h2, t3 = _dense2(t2, acc2, _cnt_col(cnt2), W2r, W2n, _row(b2), _row(g2),
                   _row(be2), W3n)

  acc3, cnt3 = _seg_sum(_W3)(t3, s3, d3,
                             jnp.zeros((_NPAD, _W3), jnp.float32))
  return _dense3(h2, acc3, _cnt_col(cnt3), W3r, _row(b3), _row(g3),
                 _row(be3), Wl, _row(bl))
